# baseline (speedup 1.0000x reference)
import functools

import jax
import jax.numpy as jnp
from jax.experimental import pallas as pl
from jax.experimental.pallas import tpu as pltpu

N = 10000
E = 160000
NFEAT = 256
DH = 512
NC_OUT = 64

ROWS = 1000


def _mlp_body(x_ref, w1_ref, b1_ref, w2_ref, b2_ref, o_ref):
    t = jnp.dot(x_ref[...], w1_ref[...], preferred_element_type=jnp.float32)
    t = jnp.maximum(t + b1_ref[...], 0.0)
    u = jnp.dot(t, w2_ref[...], preferred_element_type=jnp.float32)
    o_ref[...] = jnp.maximum(u + b2_ref[...], 0.0)


def _mlp(hpre, w1, b1, w2, b2):
    din = hpre.shape[1]
    return pl.pallas_call(
        _mlp_body,
        grid=(N // ROWS,),
        in_specs=[
            pl.BlockSpec((ROWS, din), lambda i: (i, 0)),
            pl.BlockSpec((din, DH), lambda i: (0, 0)),
            pl.BlockSpec((1, DH), lambda i: (0, 0)),
            pl.BlockSpec((DH, DH), lambda i: (0, 0)),
            pl.BlockSpec((1, DH), lambda i: (0, 0)),
        ],
        out_specs=pl.BlockSpec((ROWS, DH), lambda i: (i, 0)),
        out_shape=jax.ShapeDtypeStruct((N, DH), jnp.float32),
    )(hpre, w1, b1, w2, b2)


def _head_body(h1_ref, h2_ref, h3_ref, wa_ref, wb_ref, wc_ref, b1_ref,
               w2_ref, b2_ref, o_ref):
    u = jnp.dot(h1_ref[...], wa_ref[...], preferred_element_type=jnp.float32)
    u += jnp.dot(h2_ref[...], wb_ref[...], preferred_element_type=jnp.float32)
    u += jnp.dot(h3_ref[...], wc_ref[...], preferred_element_type=jnp.float32)
    u = jnp.maximum(u + b1_ref[...], 0.0)
    v = jnp.dot(u, w2_ref[...], preferred_element_type=jnp.float32)
    v = v + b2_ref[...]
    m = jnp.max(v, axis=1, keepdims=True)
    lse = jnp.log(jnp.sum(jnp.exp(v - m), axis=1, keepdims=True)) + m
    o_ref[...] = v - lse


def _head(h1, h2, h3, lin1_W, lin1_b, lin2_W, lin2_b):
    wa = lin1_W[0:DH]
    wb = lin1_W[DH:2 * DH]
    wc = lin1_W[2 * DH:3 * DH]
    return pl.pallas_call(
        _head_body,
        grid=(N // ROWS,),
        in_specs=[
            pl.BlockSpec((ROWS, DH), lambda i: (i, 0)),
            pl.BlockSpec((ROWS, DH), lambda i: (i, 0)),
            pl.BlockSpec((ROWS, DH), lambda i: (i, 0)),
            pl.BlockSpec((DH, 3 * DH), lambda i: (0, 0)),
            pl.BlockSpec((DH, 3 * DH), lambda i: (0, 0)),
            pl.BlockSpec((DH, 3 * DH), lambda i: (0, 0)),
            pl.BlockSpec((1, 3 * DH), lambda i: (0, 0)),
            pl.BlockSpec((3 * DH, NC_OUT), lambda i: (0, 0)),
            pl.BlockSpec((1, NC_OUT), lambda i: (0, 0)),
        ],
        out_specs=pl.BlockSpec((ROWS, NC_OUT), lambda i: (i, 0)),
        out_shape=jax.ShapeDtypeStruct((N, NC_OUT), jnp.float32),
    )(h1, h2, h3, wa, wb, wc, lin1_b.reshape(1, -1), lin2_W,
      lin2_b.reshape(1, -1))


def _aggregate(h, src, dst):
    return h + jnp.zeros_like(h).at[dst].add(h[src])


def _fold_bn(w1, b1, gamma, beta, mean, var):
    s = gamma * jax.lax.rsqrt(var + 1e-5)
    return w1 * s[None, :], ((b1 - mean) * s + beta).reshape(1, -1)


def kernel(x, edge_index, PvT, c1_W1, c1_b1, c1_gamma, c1_beta, c1_mean, c1_var, c1_W2, c1_b2, c2_W1, c2_b1, c2_gamma, c2_beta, c2_mean, c2_var, c2_W2, c2_b2, c3_W1, c3_b1, c3_gamma, c3_beta, c3_mean, c3_var, c3_W2, c3_b2, lin1_W, lin1_b, lin2_W, lin2_b):
    src = edge_index[0]
    dst = edge_index[1]

    w1a, b1a = _fold_bn(c1_W1, c1_b1, c1_gamma, c1_beta, c1_mean, c1_var)
    w1b, b1b = _fold_bn(c2_W1, c2_b1, c2_gamma, c2_beta, c2_mean, c2_var)
    w1c, b1c = _fold_bn(c3_W1, c3_b1, c3_gamma, c3_beta, c3_mean, c3_var)

    h1 = _mlp(_aggregate(x, src, dst), w1a, b1a, c1_W2, c1_b2.reshape(1, -1))
    h2 = _mlp(_aggregate(h1, src, dst), w1b, b1b, c2_W2, c2_b2.reshape(1, -1))
    h3 = _mlp(_aggregate(h2, src, dst), w1c, b1c, c3_W2, c3_b2.reshape(1, -1))
    return _head(h1, h2, h3, lin1_W, lin1_b, lin2_W, lin2_b)


# speedup vs baseline: 1.7865x; 1.7865x over previous
import functools

import jax
import jax.numpy as jnp
from jax import lax
from jax.experimental import pallas as pl
from jax.experimental.pallas import tpu as pltpu
from jax.experimental.pallas import tpu_sc as plsc

N = 10000
E = 160000
NFEAT = 256
DH = 512
NC_OUT = 64

NPAD = 10240
NTILES = 16
ROWS = 640
ROWS_H = 400

NW = 32
BLK = 4000
NBLK = E // BLK
PACK = 16384


def _make_agg(d, rounds):
    R = NPAD // (NW * rounds)
    mesh = plsc.VectorSubcoreMesh(
        core_axis_name="c", subcore_axis_name="s", num_cores=2,
        num_subcores=NTILES)

    @functools.partial(
        pl.kernel,
        mesh=mesh,
        compiler_params=pltpu.CompilerParams(needs_layout_passes=False),
        out_type=jax.ShapeDtypeStruct((NPAD, d), jnp.float32),
        scratch_types=[
            pltpu.VMEM((BLK,), jnp.int32),
            pltpu.VMEM((BLK,), jnp.int32),
            pltpu.VMEM((BLK + 32,), jnp.int32),
            pltpu.VMEM((16, d), jnp.float32),
            pltpu.VMEM((16, d), jnp.float32),
            pltpu.VMEM((R + 1, d), jnp.float32),
            pltpu.SemaphoreType.DMA,
            pltpu.SemaphoreType.DMA,
            pltpu.SemaphoreType.DMA,
            pltpu.SemaphoreType.DMA,
        ],
    )
    def agg(h_hbm, ep_hbm, out_hbm, eb0, eb1, sel, rows0, rows1, acc,
            esem0, esem1, gsem0, gsem1):
        c = lax.axis_index("c")
        s = lax.axis_index("s")
        w = s * 2 + c

        def accumulate(rows_v, base_e, lo):
            def rbody(j, cc):
                dj = lax.shift_right_logical(
                    sel[pl.ds(base_e + j, 16)][0], 14) - lo
                vals = [rows_v[j, pl.ds(k * 16, 16)] for k in range(d // 16)]
                for k in range(d // 16):
                    plsc.addupdate(acc.at[dj, pl.ds(k * 16, 16)], vals[k])
                return cc
            lax.fori_loop(0, 16, rbody, jnp.int32(0))

        def start_gather(g, rows_v, gsem):
            sidx = sel[pl.ds(g * 16, 16)] & (PACK - 1)
            pltpu.async_copy(h_hbm.at[sidx], rows_v, gsem)

        def process_block(eb, lo):
            def fbody(i, cur):
                e16 = eb[pl.ds(i * 16, 16)]
                m = (e16 >= lo * PACK) & (e16 < (lo + R) * PACK)
                mi = jnp.where(m, 1, 0)
                pc = plsc.cumsum(mi)
                pos = cur + pc - mi
                plsc.store_scatter(sel, [pos], e16, mask=m)
                return cur + lax.rev(pc, (0,))[0]

            cnt = lax.fori_loop(0, BLK // 16, fbody, jnp.int32(0))
            sel[pl.ds(cnt, 16)] = jnp.full((16,), (lo + R) * PACK, jnp.int32)

            nb = jnp.maximum((cnt + 15) // 16, 1)
            start_gather(0, rows0, gsem0)

            def gpair(gp, carry2):
                g0 = gp * 2
                g1 = g0 + 1

                @pl.when(g1 < nb)
                def _():
                    start_gather(g1, rows1, gsem1)
                pltpu.make_async_copy(h_hbm.at[pl.ds(0, 16)], rows0,
                                      gsem0).wait()
                accumulate(rows0, g0 * 16, lo)

                @pl.when(g0 + 2 < nb)
                def _():
                    start_gather(g0 + 2, rows0, gsem0)

                @pl.when(g1 < nb)
                def _():
                    pltpu.make_async_copy(h_hbm.at[pl.ds(0, 16)], rows1,
                                          gsem1).wait()
                    accumulate(rows1, g1 * 16, lo)
                return carry2

            lax.fori_loop(0, (nb + 1) // 2, gpair, jnp.int32(0))

        for r in range(rounds):
            lo = (r * NW + w) * R
            pltpu.sync_copy(h_hbm.at[pl.ds(lo, R)], acc.at[pl.ds(0, R)])
            pltpu.async_copy(ep_hbm.at[pl.ds(0, BLK)], eb0, esem0)

            def pair_body(pb, carry):
                b0 = pb * 2
                pltpu.async_copy(ep_hbm.at[pl.ds((b0 + 1) * BLK, BLK)],
                                 eb1, esem1)
                pltpu.make_async_copy(ep_hbm.at[pl.ds(0, BLK)], eb0,
                                      esem0).wait()
                process_block(eb0, lo)

                @pl.when(b0 + 2 < NBLK)
                def _():
                    pltpu.async_copy(ep_hbm.at[pl.ds((b0 + 2) * BLK, BLK)],
                                     eb0, esem0)
                pltpu.make_async_copy(ep_hbm.at[pl.ds(0, BLK)], eb1,
                                      esem1).wait()
                process_block(eb1, lo)
                return carry

            lax.fori_loop(0, NBLK // 2, pair_body, jnp.int32(0))
            pltpu.sync_copy(acc.at[pl.ds(0, R)], out_hbm.at[pl.ds(lo, R)])

    return agg


def _mlp_body(x_ref, w1_ref, b1_ref, w2_ref, b2_ref, o_ref):
    t = jnp.dot(x_ref[...], w1_ref[...], preferred_element_type=jnp.float32)
    t = jnp.maximum(t + b1_ref[...], 0.0)
    u = jnp.dot(t, w2_ref[...], preferred_element_type=jnp.float32)
    o_ref[...] = jnp.maximum(u + b2_ref[...], 0.0)


def _mlp(hpre, w1, b1, w2, b2):
    din = hpre.shape[1]
    return pl.pallas_call(
        _mlp_body,
        grid=(NPAD // ROWS,),
        in_specs=[
            pl.BlockSpec((ROWS, din), lambda i: (i, 0)),
            pl.BlockSpec((din, DH), lambda i: (0, 0)),
            pl.BlockSpec((1, DH), lambda i: (0, 0)),
            pl.BlockSpec((DH, DH), lambda i: (0, 0)),
            pl.BlockSpec((1, DH), lambda i: (0, 0)),
        ],
        out_specs=pl.BlockSpec((ROWS, DH), lambda i: (i, 0)),
        out_shape=jax.ShapeDtypeStruct((NPAD, DH), jnp.float32),
    )(hpre, w1, b1, w2, b2)


def _head_body(h1_ref, h2_ref, h3_ref, wa_ref, wb_ref, wc_ref, b1_ref,
               w2_ref, b2_ref, o_ref):
    u = jnp.dot(h1_ref[...], wa_ref[...], preferred_element_type=jnp.float32)
    u += jnp.dot(h2_ref[...], wb_ref[...], preferred_element_type=jnp.float32)
    u += jnp.dot(h3_ref[...], wc_ref[...], preferred_element_type=jnp.float32)
    u = jnp.maximum(u + b1_ref[...], 0.0)
    v = jnp.dot(u, w2_ref[...], preferred_element_type=jnp.float32)
    v = v + b2_ref[...]
    m = jnp.max(v, axis=1, keepdims=True)
    lse = jnp.log(jnp.sum(jnp.exp(v - m), axis=1, keepdims=True)) + m
    o_ref[...] = v - lse


def _head(h1, h2, h3, lin1_W, lin1_b, lin2_W, lin2_b):
    wa = lin1_W[0:DH]
    wb = lin1_W[DH:2 * DH]
    wc = lin1_W[2 * DH:3 * DH]
    return pl.pallas_call(
        _head_body,
        grid=(N // ROWS_H,),
        in_specs=[
            pl.BlockSpec((ROWS_H, DH), lambda i: (i, 0)),
            pl.BlockSpec((ROWS_H, DH), lambda i: (i, 0)),
            pl.BlockSpec((ROWS_H, DH), lambda i: (i, 0)),
            pl.BlockSpec((DH, 3 * DH), lambda i: (0, 0)),
            pl.BlockSpec((DH, 3 * DH), lambda i: (0, 0)),
            pl.BlockSpec((DH, 3 * DH), lambda i: (0, 0)),
            pl.BlockSpec((1, 3 * DH), lambda i: (0, 0)),
            pl.BlockSpec((3 * DH, NC_OUT), lambda i: (0, 0)),
            pl.BlockSpec((1, NC_OUT), lambda i: (0, 0)),
        ],
        out_specs=pl.BlockSpec((ROWS_H, NC_OUT), lambda i: (i, 0)),
        out_shape=jax.ShapeDtypeStruct((N, NC_OUT), jnp.float32),
    )(h1, h2, h3, wa, wb, wc, lin1_b.reshape(1, -1), lin2_W,
      lin2_b.reshape(1, -1))


def _fold_bn(w1, b1, gamma, beta, mean, var):
    s = gamma * lax.rsqrt(var + 1e-5)
    return w1 * s[None, :], ((b1 - mean) * s + beta).reshape(1, -1)


def kernel(x, edge_index, PvT, c1_W1, c1_b1, c1_gamma, c1_beta, c1_mean, c1_var, c1_W2, c1_b2, c2_W1, c2_b1, c2_gamma, c2_beta, c2_mean, c2_var, c2_W2, c2_b2, c3_W1, c3_b1, c3_gamma, c3_beta, c3_mean, c3_var, c3_W2, c3_b2, lin1_W, lin1_b, lin2_W, lin2_b):
    src = edge_index[0]
    dst = edge_index[1]
    epack = dst * PACK + src

    w1a, b1a = _fold_bn(c1_W1, c1_b1, c1_gamma, c1_beta, c1_mean, c1_var)
    w1b, b1b = _fold_bn(c2_W1, c2_b1, c2_gamma, c2_beta, c2_mean, c2_var)
    w1c, b1c = _fold_bn(c3_W1, c3_b1, c3_gamma, c3_beta, c3_mean, c3_var)

    agg256 = _make_agg(NFEAT, 1)
    agg512 = _make_agg(DH, 2)

    x_pad = jnp.pad(x, ((0, NPAD - N), (0, 0)))
    h1 = _mlp(agg256(x_pad, epack), w1a, b1a, c1_W2, c1_b2.reshape(1, -1))
    h2 = _mlp(agg512(h1, epack), w1b, b1b, c2_W2, c2_b2.reshape(1, -1))
    h3 = _mlp(agg512(h2, epack), w1c, b1c, c3_W2, c3_b2.reshape(1, -1))
    return _head(h1, h2, h3, lin1_W, lin1_b, lin2_W, lin2_b)
